# initial kernel scaffold (unmeasured)
import jax
import jax.numpy as jnp
from jax import lax
from jax.experimental import pallas as pl
from jax.experimental.pallas import tpu as pltpu

N_DEV = 4
B, SQ, D = 1, 2048, 1024
HQ_LOCAL = 8
DH = 128
SCALE = 0.08838834764831843
CHUNK = SQ // N_DEV


def _attn_body(x_ref, wq_ref, wk_ref, wv_ref, wo_ref, cos_ref, sin_ref, out_ref):
    h = pl.program_id(0)
    x = x_ref[...]
    q = jnp.dot(x, wq_ref[...], preferred_element_type=jnp.float32)
    k = jnp.dot(x, wk_ref[...], preferred_element_type=jnp.float32)
    v = jnp.dot(x, wv_ref[...], preferred_element_type=jnp.float32)

    row = lax.broadcasted_iota(jnp.int32, (DH, DH), 0)
    col = lax.broadcasted_iota(jnp.int32, (DH, DH), 1)
    rmat = jnp.where((col % 2 == 0) & (row == col + 1), -1.0, 0.0) + jnp.where(
        (col % 2 == 1) & (row == col - 1), 1.0, 0.0
    )
    cos = cos_ref[...]
    sin = sin_ref[...]
    q = q * cos + jnp.dot(q, rmat, preferred_element_type=jnp.float32) * sin
    k = k * cos + jnp.dot(k, rmat, preferred_element_type=jnp.float32) * sin

    s = (
        lax.dot_general(q, k, (((1,), (1,)), ((), ())),
                        preferred_element_type=jnp.float32)
        * SCALE
    )
    m = jnp.max(s, axis=1, keepdims=True)
    w = jnp.exp(s - m)
    w = w / jnp.sum(w, axis=1, keepdims=True)
    ctx = jnp.dot(w, v, preferred_element_type=jnp.float32)
    part = jnp.dot(ctx, wo_ref[...], preferred_element_type=jnp.float32)

    @pl.when(h == 0)
    def _():
        out_ref[...] = part

    @pl.when(h > 0)
    def _():
        out_ref[...] = out_ref[...] + part


def _allreduce_body(p_ref, out_ref, rs_recv, rs_acc, send_sems, recv_sems):
    my = lax.axis_index("i")
    left = lax.rem(my + N_DEV - 1, N_DEV)
    right = lax.rem(my + 1, N_DEV)

    barrier_sem = pltpu.get_barrier_semaphore()
    for nbr in (left, right):
        pl.semaphore_signal(
            barrier_sem, inc=1,
            device_id=(nbr,), device_id_type=pl.DeviceIdType.MESH,
        )
    pl.semaphore_wait(barrier_sem, 2)

    for s in range(N_DEV - 1):
        send_c = lax.rem(my + 2 * N_DEV - 1 - s, N_DEV)
        recv_c = lax.rem(my + 2 * N_DEV - 2 - s, N_DEV)
        if s == 0:
            src = p_ref.at[pl.ds(send_c * CHUNK, CHUNK), :]
        else:
            src = rs_acc.at[s - 1]
        rdma = pltpu.make_async_remote_copy(
            src_ref=src,
            dst_ref=rs_recv.at[s],
            send_sem=send_sems.at[s],
            recv_sem=recv_sems.at[s],
            device_id=(right,),
            device_id_type=pl.DeviceIdType.MESH,
        )
        rdma.start()
        rdma.wait()
        acc = rs_recv[s] + p_ref[pl.ds(recv_c * CHUNK, CHUNK), :]
        if s < N_DEV - 2:
            rs_acc[s] = acc
        else:
            out_ref[pl.ds(my * CHUNK, CHUNK), :] = acc

    for s in range(N_DEV - 1):
        send_c = lax.rem(my + 2 * N_DEV - s, N_DEV)
        rdma = pltpu.make_async_remote_copy(
            src_ref=out_ref.at[pl.ds(send_c * CHUNK, CHUNK), :],
            dst_ref=out_ref.at[pl.ds(send_c * CHUNK, CHUNK), :],
            send_sem=send_sems.at[N_DEV - 1 + s],
            recv_sem=recv_sems.at[N_DEV - 1 + s],
            device_id=(right,),
            device_id_type=pl.DeviceIdType.MESH,
        )
        rdma.start()
        rdma.wait()


def kernel(x, Wq, Wk, Wv, Wo):
    x2 = x.reshape(SQ, D)

    inv = 1.0 / (10000.0 ** (jnp.arange(0, DH, 2, dtype=jnp.float32) / DH))
    pos = jnp.arange(SQ, dtype=jnp.float32)[:, None] * inv[None, :]
    cos = jnp.repeat(jnp.cos(pos), 2, axis=-1)
    sin = jnp.repeat(jnp.sin(pos), 2, axis=-1)

    partial = pl.pallas_call(
        _attn_body,
        grid=(HQ_LOCAL,),
        in_specs=[
            pl.BlockSpec((SQ, D), lambda h: (0, 0)),
            pl.BlockSpec((D, DH), lambda h: (0, h)),
            pl.BlockSpec((D, DH), lambda h: (0, h)),
            pl.BlockSpec((D, DH), lambda h: (0, h)),
            pl.BlockSpec((DH, D), lambda h: (h, 0)),
            pl.BlockSpec((SQ, DH), lambda h: (0, 0)),
            pl.BlockSpec((SQ, DH), lambda h: (0, 0)),
        ],
        out_specs=pl.BlockSpec((SQ, D), lambda h: (0, 0)),
        out_shape=jax.ShapeDtypeStruct((SQ, D), jnp.float32),
    )(x2, Wq, Wk, Wv, Wo, cos, sin)

    out = pl.pallas_call(
        _allreduce_body,
        out_shape=jax.ShapeDtypeStruct((SQ, D), jnp.float32),
        in_specs=[pl.BlockSpec(memory_space=pltpu.VMEM)],
        out_specs=pl.BlockSpec(memory_space=pltpu.VMEM),
        scratch_shapes=[
            pltpu.VMEM((N_DEV - 1, CHUNK, D), jnp.float32),
            pltpu.VMEM((N_DEV - 2, CHUNK, D), jnp.float32),
            pltpu.SemaphoreType.DMA((2 * (N_DEV - 1),)),
            pltpu.SemaphoreType.DMA((2 * (N_DEV - 1),)),
        ],
        compiler_params=pltpu.CompilerParams(collective_id=0),
    )(partial)

    return out.reshape(B, SQ, D)


# baseline (device time: 304371 ns/iter reference)
import jax
import jax.numpy as jnp
from jax import lax
from jax.experimental import pallas as pl
from jax.experimental.pallas import tpu as pltpu

N_DEV = 4
B, SQ, D = 1, 2048, 1024
HQ_LOCAL = 8
DH = 128
SCALE = 0.08838834764831843
CHUNK = SQ // N_DEV


def _attn_body(x_ref, wq_ref, wk_ref, wv_ref, wo_ref, cos_ref, sin_ref, out_ref):
    h = pl.program_id(0)
    x = x_ref[...]
    q = jnp.dot(x, wq_ref[...], preferred_element_type=jnp.float32)
    k = jnp.dot(x, wk_ref[...], preferred_element_type=jnp.float32)
    v = jnp.dot(x, wv_ref[...], preferred_element_type=jnp.float32)

    row = lax.broadcasted_iota(jnp.int32, (DH, DH), 0)
    col = lax.broadcasted_iota(jnp.int32, (DH, DH), 1)
    rmat = jnp.where((col % 2 == 0) & (row == col + 1), -1.0, 0.0) + jnp.where(
        (col % 2 == 1) & (row == col - 1), 1.0, 0.0
    )
    cos = cos_ref[...]
    sin = sin_ref[...]
    q = q * cos + jnp.dot(q, rmat, preferred_element_type=jnp.float32) * sin
    k = k * cos + jnp.dot(k, rmat, preferred_element_type=jnp.float32) * sin

    s = (
        lax.dot_general(q, k, (((1,), (1,)), ((), ())),
                        preferred_element_type=jnp.float32)
        * SCALE
    )
    m = jnp.max(s, axis=1, keepdims=True)
    w = jnp.exp(s - m)
    w = w / jnp.sum(w, axis=1, keepdims=True)
    ctx = jnp.dot(w, v, preferred_element_type=jnp.float32)
    part = jnp.dot(ctx, wo_ref[...], preferred_element_type=jnp.float32)

    @pl.when(h == 0)
    def _():
        out_ref[...] = part

    @pl.when(h > 0)
    def _():
        out_ref[...] = out_ref[...] + part


def _allreduce_body(p_ref, out_ref, rs_recv, rs_acc, send_sems, recv_sems):
    my = lax.axis_index("i")
    left = lax.rem(my + N_DEV - 1, N_DEV)
    right = lax.rem(my + 1, N_DEV)

    barrier_sem = pltpu.get_barrier_semaphore()
    for nbr in (left, right):
        pl.semaphore_signal(
            barrier_sem, inc=1,
            device_id=(nbr,), device_id_type=pl.DeviceIdType.MESH,
        )
    pl.semaphore_wait(barrier_sem, 2)

    for s in range(N_DEV - 1):
        send_c = lax.rem(my + 2 * N_DEV - 1 - s, N_DEV)
        recv_c = lax.rem(my + 2 * N_DEV - 2 - s, N_DEV)
        if s == 0:
            src = p_ref.at[pl.ds(send_c * CHUNK, CHUNK), :]
        else:
            src = rs_acc.at[s - 1]
        rdma = pltpu.make_async_remote_copy(
            src_ref=src,
            dst_ref=rs_recv.at[s],
            send_sem=send_sems.at[s],
            recv_sem=recv_sems.at[s],
            device_id=(right,),
            device_id_type=pl.DeviceIdType.MESH,
        )
        rdma.start()
        rdma.wait()
        acc = rs_recv[s] + p_ref[pl.ds(recv_c * CHUNK, CHUNK), :]
        if s < N_DEV - 2:
            rs_acc[s] = acc
        else:
            out_ref[pl.ds(my * CHUNK, CHUNK), :] = acc

    for s in range(N_DEV - 1):
        send_c = lax.rem(my + 2 * N_DEV - s, N_DEV)
        rdma = pltpu.make_async_remote_copy(
            src_ref=out_ref.at[pl.ds(send_c * CHUNK, CHUNK), :],
            dst_ref=out_ref.at[pl.ds(send_c * CHUNK, CHUNK), :],
            send_sem=send_sems.at[N_DEV - 1 + s],
            recv_sem=recv_sems.at[N_DEV - 1 + s],
            device_id=(right,),
            device_id_type=pl.DeviceIdType.MESH,
        )
        rdma.start()
        rdma.wait()


def kernel(x, Wq, Wk, Wv, Wo):
    x2 = x.reshape(SQ, D)

    inv = 1.0 / (10000.0 ** (jnp.arange(0, DH, 2, dtype=jnp.float32) / DH))
    pos = jnp.arange(SQ, dtype=jnp.float32)[:, None] * inv[None, :]
    cos = jnp.repeat(jnp.cos(pos), 2, axis=-1)
    sin = jnp.repeat(jnp.sin(pos), 2, axis=-1)

    partial = pl.pallas_call(
        _attn_body,
        grid=(HQ_LOCAL,),
        in_specs=[
            pl.BlockSpec((SQ, D), lambda h: (0, 0)),
            pl.BlockSpec((D, DH), lambda h: (0, h)),
            pl.BlockSpec((D, DH), lambda h: (0, h)),
            pl.BlockSpec((D, DH), lambda h: (0, h)),
            pl.BlockSpec((DH, D), lambda h: (h, 0)),
            pl.BlockSpec((SQ, DH), lambda h: (0, 0)),
            pl.BlockSpec((SQ, DH), lambda h: (0, 0)),
        ],
        out_specs=pl.BlockSpec((SQ, D), lambda h: (0, 0)),
        out_shape=jax.ShapeDtypeStruct((SQ, D), jnp.float32),
        compiler_params=pltpu.CompilerParams(
            vmem_limit_bytes=60 * 1024 * 1024,
        ),
    )(x2, Wq, Wk, Wv, Wo, cos, sin)

    out = pl.pallas_call(
        _allreduce_body,
        out_shape=jax.ShapeDtypeStruct((SQ, D), jnp.float32),
        in_specs=[pl.BlockSpec(memory_space=pltpu.VMEM)],
        out_specs=pl.BlockSpec(memory_space=pltpu.VMEM),
        scratch_shapes=[
            pltpu.VMEM((N_DEV - 1, CHUNK, D), jnp.float32),
            pltpu.VMEM((N_DEV - 2, CHUNK, D), jnp.float32),
            pltpu.SemaphoreType.DMA((2 * (N_DEV - 1),)),
            pltpu.SemaphoreType.DMA((2 * (N_DEV - 1),)),
        ],
        compiler_params=pltpu.CompilerParams(collective_id=0),
    )(partial)

    return out.reshape(B, SQ, D)


# device time: 187520 ns/iter; 1.6231x vs baseline; 1.6231x over previous
import jax
import jax.numpy as jnp
from jax import lax
from jax.experimental import pallas as pl
from jax.experimental.pallas import tpu as pltpu

N_DEV = 4
B, SQ, D = 1, 2048, 1024
HQ_LOCAL = 8
DH = 128
SCALE = 0.08838834764831843
CHUNK = SQ // N_DEV


NPAIR = HQ_LOCAL // 2
PW = 2 * DH


def _attn_body(x_ref, wq_ref, wk_ref, wv_ref, wo_ref, cos_ref, sin_ref, out_ref):
    pair = pl.program_id(0)
    x = x_ref[...]
    q = jnp.dot(x, wq_ref[...], preferred_element_type=jnp.float32)
    k = jnp.dot(x, wk_ref[...], preferred_element_type=jnp.float32)
    v = jnp.dot(x, wv_ref[...], preferred_element_type=jnp.float32)

    row = lax.broadcasted_iota(jnp.int32, (PW, PW), 0)
    col = lax.broadcasted_iota(jnp.int32, (PW, PW), 1)
    rmat = jnp.where((col % 2 == 0) & (row == col + 1), -1.0, 0.0) + jnp.where(
        (col % 2 == 1) & (row == col - 1), 1.0, 0.0
    )
    cos = cos_ref[...]
    sin = sin_ref[...]
    q = q * cos + jnp.dot(q, rmat, preferred_element_type=jnp.float32) * sin
    k = k * cos + jnp.dot(k, rmat, preferred_element_type=jnp.float32) * sin

    ctxs = []
    for j in range(2):
        sl = slice(j * DH, (j + 1) * DH)
        s = lax.dot_general(
            q[:, sl], k[:, sl], (((1,), (1,)), ((), ())),
            preferred_element_type=jnp.float32,
        )
        w = jnp.exp(s)
        r = 1.0 / jnp.sum(w, axis=1, keepdims=True)
        ctxs.append(
            jnp.dot(w, v[:, sl], preferred_element_type=jnp.float32) * r
        )
    part = jnp.dot(
        jnp.concatenate(ctxs, axis=1), wo_ref[...],
        preferred_element_type=jnp.float32,
    )

    @pl.when(pair == 0)
    def _():
        out_ref[...] = part

    @pl.when(pair > 0)
    def _():
        out_ref[...] = out_ref[...] + part


HALF = CHUNK // 2


def _allreduce_body(
    p_ref, out_ref,
    recv_cw, recv_ccw, acc_cw, acc_ccw,
    ss_cw, rs_cw, ss_ccw, rs_ccw,
):
    my = lax.axis_index("i")
    left = lax.rem(my + N_DEV - 1, N_DEV)
    right = lax.rem(my + 1, N_DEV)

    barrier_sem = pltpu.get_barrier_semaphore()
    for nbr in (left, right):
        pl.semaphore_signal(
            barrier_sem, inc=1,
            device_id=(nbr,), device_id_type=pl.DeviceIdType.MESH,
        )
    pl.semaphore_wait(barrier_sem, 2)

    def hop(src, dst, sems_s, sems_r, s, target):
        rdma = pltpu.make_async_remote_copy(
            src_ref=src, dst_ref=dst,
            send_sem=sems_s.at[s], recv_sem=sems_r.at[s],
            device_id=(target,), device_id_type=pl.DeviceIdType.MESH,
        )
        rdma.start()
        return rdma

    for s in range(N_DEV - 1):
        send_c = lax.rem(my + 2 * N_DEV - 1 - s, N_DEV)
        recv_c = lax.rem(my + 2 * N_DEV - 2 - s, N_DEV)
        send_cc = lax.rem(my + 1 + s, N_DEV)
        recv_cc = lax.rem(my + 2 + s, N_DEV)
        if s == 0:
            src_cw = p_ref.at[pl.ds(send_c * CHUNK, HALF), :]
            src_cc = p_ref.at[pl.ds(send_cc * CHUNK + HALF, HALF), :]
        else:
            src_cw = acc_cw.at[s - 1]
            src_cc = acc_ccw.at[s - 1]
        r1 = hop(src_cw, recv_cw.at[s], ss_cw, rs_cw, s, right)
        r2 = hop(src_cc, recv_ccw.at[s], ss_ccw, rs_ccw, s, left)
        r1.wait()
        r2.wait()
        a_cw = recv_cw[s] + p_ref[pl.ds(recv_c * CHUNK, HALF), :]
        a_cc = recv_ccw[s] + p_ref[pl.ds(recv_cc * CHUNK + HALF, HALF), :]
        if s < N_DEV - 2:
            acc_cw[s] = a_cw
            acc_ccw[s] = a_cc
        else:
            out_ref[pl.ds(my * CHUNK, HALF), :] = a_cw
            out_ref[pl.ds(my * CHUNK + HALF, HALF), :] = a_cc

    for s in range(N_DEV - 1):
        send_c = lax.rem(my + 2 * N_DEV - s, N_DEV)
        send_cc = lax.rem(my + s, N_DEV)
        src_cw = out_ref.at[pl.ds(send_c * CHUNK, HALF), :]
        src_cc = out_ref.at[pl.ds(send_cc * CHUNK + HALF, HALF), :]
        r1 = hop(src_cw, src_cw, ss_cw, rs_cw, N_DEV - 1 + s, right)
        r2 = hop(src_cc, src_cc, ss_ccw, rs_ccw, N_DEV - 1 + s, left)
        r1.wait()
        r2.wait()


def kernel(x, Wq, Wk, Wv, Wo):
    x2 = x.reshape(SQ, D)

    inv = 1.0 / (10000.0 ** (jnp.arange(0, DH, 2, dtype=jnp.float32) / DH))
    pos = jnp.arange(SQ, dtype=jnp.float32)[:, None] * inv[None, :]
    cos = jnp.tile(jnp.repeat(jnp.cos(pos), 2, axis=-1), (1, 2))
    sin = jnp.tile(jnp.repeat(jnp.sin(pos), 2, axis=-1), (1, 2))

    partial = pl.pallas_call(
        _attn_body,
        grid=(NPAIR,),
        in_specs=[
            pl.BlockSpec((SQ, D), lambda h: (0, 0)),
            pl.BlockSpec((D, PW), lambda h: (0, h)),
            pl.BlockSpec((D, PW), lambda h: (0, h)),
            pl.BlockSpec((D, PW), lambda h: (0, h)),
            pl.BlockSpec((PW, D), lambda h: (h, 0)),
            pl.BlockSpec((SQ, PW), lambda h: (0, 0)),
            pl.BlockSpec((SQ, PW), lambda h: (0, 0)),
        ],
        out_specs=pl.BlockSpec((SQ, D), lambda h: (0, 0)),
        out_shape=jax.ShapeDtypeStruct((SQ, D), jnp.float32),
        compiler_params=pltpu.CompilerParams(
            vmem_limit_bytes=60 * 1024 * 1024,
        ),
    )(x2, Wq * SCALE, Wk, Wv, Wo, cos, sin)

    out = pl.pallas_call(
        _allreduce_body,
        out_shape=jax.ShapeDtypeStruct((SQ, D), jnp.float32),
        in_specs=[pl.BlockSpec(memory_space=pltpu.VMEM)],
        out_specs=pl.BlockSpec(memory_space=pltpu.VMEM),
        scratch_shapes=[
            pltpu.VMEM((N_DEV - 1, HALF, D), jnp.float32),
            pltpu.VMEM((N_DEV - 1, HALF, D), jnp.float32),
            pltpu.VMEM((N_DEV - 2, HALF, D), jnp.float32),
            pltpu.VMEM((N_DEV - 2, HALF, D), jnp.float32),
            pltpu.SemaphoreType.DMA((2 * (N_DEV - 1),)),
            pltpu.SemaphoreType.DMA((2 * (N_DEV - 1),)),
            pltpu.SemaphoreType.DMA((2 * (N_DEV - 1),)),
            pltpu.SemaphoreType.DMA((2 * (N_DEV - 1),)),
        ],
        compiler_params=pltpu.CompilerParams(collective_id=0),
    )(partial)

    return out.reshape(B, SQ, D)


# device time: 127137 ns/iter; 2.3940x vs baseline; 1.4749x over previous
import jax
import jax.numpy as jnp
from jax import lax
from jax.experimental import pallas as pl
from jax.experimental.pallas import tpu as pltpu

N_DEV = 4
B, SQ, D = 1, 2048, 1024
HQ_LOCAL = 8
DH = 128
NPAIR = HQ_LOCAL // 2
PW = 2 * DH
SCALE = 0.08838834764831843
CHUNK = SQ // N_DEV
HALF = CHUNK // 2


def _fused_body(
    x_ref, wq_ref, wk_ref, wv_ref, wo_ref, cos_ref, sin_ref, out_ref,
    kb, vb, rs_recv_cw, rs_recv_ccw, sb_cw, sb_ccw, ag_cw, ag_ccw,
    ss_cw, rs_cw, ss_ccw, rs_ccw,
):
    f32 = jnp.float32
    bf16 = jnp.bfloat16
    my = lax.axis_index("i")
    left = lax.rem(my + N_DEV - 1, N_DEV)
    right = lax.rem(my + 1, N_DEV)

    barrier_sem = pltpu.get_barrier_semaphore()
    for nbr in (left, right):
        pl.semaphore_signal(
            barrier_sem, inc=1,
            device_id=(nbr,), device_id_type=pl.DeviceIdType.MESH,
        )
    pl.semaphore_wait(barrier_sem, 2)

    row = lax.broadcasted_iota(jnp.int32, (PW, PW), 0)
    col = lax.broadcasted_iota(jnp.int32, (PW, PW), 1)
    rmat = jnp.where((col % 2 == 0) & (row == col + 1), -1.0, 0.0) + jnp.where(
        (col % 2 == 1) & (row == col - 1), 1.0, 0.0
    )

    def rope2(t, c, s):
        cc = jnp.concatenate([c, c], axis=1)
        ss = jnp.concatenate([s, s], axis=1)
        return t * cc + jnp.dot(t, rmat, preferred_element_type=f32) * ss

    cos_all = cos_ref[...]
    sin_all = sin_ref[...]
    for p in range(NPAIR):
        csl = slice(p * PW, (p + 1) * PW)
        k = jnp.dot(x_ref[...], wk_ref[:, csl], preferred_element_type=f32)
        kb[:, csl] = rope2(k, cos_all, sin_all).astype(bf16)
        v = jnp.dot(x_ref[...], wv_ref[:, csl], preferred_element_type=f32)
        vb[:, csl] = v.astype(bf16)

    def compute_chunk(c):
        rows = pl.ds(c * CHUNK, CHUNK)
        xr = x_ref[rows, :]
        cosr = cos_ref[rows, :]
        sinr = sin_ref[rows, :]
        acc = None
        for p in range(NPAIR):
            csl = slice(p * PW, (p + 1) * PW)
            q = jnp.dot(xr, wq_ref[:, csl], preferred_element_type=f32)
            q = rope2(q * SCALE, cosr, sinr)
            qb = q.astype(bf16)
            ctxs = []
            for j in range(2):
                hsl = slice(p * PW + j * DH, p * PW + (j + 1) * DH)
                jsl = slice(j * DH, (j + 1) * DH)
                s = lax.dot_general(
                    qb[:, jsl], kb[:, hsl], (((1,), (1,)), ((), ())),
                    preferred_element_type=f32,
                )
                w = jnp.exp(s)
                r = 1.0 / jnp.sum(w, axis=1, keepdims=True)
                ctxs.append(
                    jnp.dot(w.astype(bf16), vb[:, hsl],
                            preferred_element_type=f32) * r
                )
            part = jnp.dot(
                jnp.concatenate(ctxs, axis=1), wo_ref[csl, :],
                preferred_element_type=f32,
            )
            acc = part if acc is None else acc + part
        out_ref[rows, :] = acc
        return acc

    def rdma(src, dst, ss, rs, idx, target):
        r = pltpu.make_async_remote_copy(
            src_ref=src, dst_ref=dst,
            send_sem=ss.at[idx], recv_sem=rs.at[idx],
            device_id=(target,), device_id_type=pl.DeviceIdType.MESH,
        )
        r.start()
        return r

    c0 = lax.rem(my + N_DEV - 1, N_DEV)
    c1 = lax.rem(my + 1, N_DEV)
    c2 = lax.rem(my + 2, N_DEV)

    a0 = compute_chunk(c0)
    sb_cw[0] = a0[:HALF, :].astype(bf16)
    r_cw0 = rdma(sb_cw.at[0], rs_recv_cw.at[0], ss_cw, rs_cw, 0, right)

    a1 = compute_chunk(c1)
    sb_ccw[0] = a1[HALF:, :].astype(bf16)
    r_ccw0 = rdma(sb_ccw.at[0], rs_recv_ccw.at[0], ss_ccw, rs_ccw, 0, left)

    a2 = compute_chunk(c2)
    r_cw0.wait_recv()
    sb_cw[1] = (rs_recv_cw[0].astype(f32) + a2[:HALF, :]).astype(bf16)
    r_cw1 = rdma(sb_cw.at[1], rs_recv_cw.at[1], ss_cw, rs_cw, 1, right)
    r_ccw0.wait_recv()
    sb_ccw[1] = (rs_recv_ccw[0].astype(f32) + a2[HALF:, :]).astype(bf16)
    r_ccw1 = rdma(sb_ccw.at[1], rs_recv_ccw.at[1], ss_ccw, rs_ccw, 1, left)

    a3 = compute_chunk(my)
    r_cw1.wait_recv()
    sb_cw[2] = (
        rs_recv_cw[1].astype(f32) + out_ref[pl.ds(c1 * CHUNK, HALF), :]
    ).astype(bf16)
    r_cw2 = rdma(sb_cw.at[2], rs_recv_cw.at[2], ss_cw, rs_cw, 2, right)
    r_ccw1.wait_recv()
    sb_ccw[2] = (
        rs_recv_ccw[1].astype(f32)
        + out_ref[pl.ds(c0 * CHUNK + HALF, HALF), :]
    ).astype(bf16)
    r_ccw2 = rdma(sb_ccw.at[2], rs_recv_ccw.at[2], ss_ccw, rs_ccw, 2, left)

    r_cw2.wait_recv()
    fin_low = rs_recv_cw[2].astype(f32) + a3[:HALF, :]
    out_ref[pl.ds(my * CHUNK, HALF), :] = fin_low
    ag_cw[0] = fin_low.astype(bf16)
    r_ccw2.wait_recv()
    fin_high = rs_recv_ccw[2].astype(f32) + a3[HALF:, :]
    out_ref[pl.ds(my * CHUNK + HALF, HALF), :] = fin_high
    ag_ccw[0] = fin_high.astype(bf16)

    started = [r_cw0, r_ccw0, r_cw1, r_ccw1, r_cw2, r_ccw2]
    for s in range(N_DEV - 1):
        recv_c = lax.rem(my + 2 * N_DEV - 1 - s, N_DEV)
        recv_cc = lax.rem(my + 1 + s, N_DEV)
        g1 = rdma(ag_cw.at[s], ag_cw.at[s + 1], ss_cw, rs_cw,
                  N_DEV - 1 + s, right)
        g2 = rdma(ag_ccw.at[s], ag_ccw.at[s + 1], ss_ccw, rs_ccw,
                  N_DEV - 1 + s, left)
        g1.wait_recv()
        out_ref[pl.ds(recv_c * CHUNK, HALF), :] = ag_cw[s + 1].astype(f32)
        g2.wait_recv()
        out_ref[pl.ds(recv_cc * CHUNK + HALF, HALF), :] = ag_ccw[
            s + 1
        ].astype(f32)
        started.extend([g1, g2])

    for r in started:
        r.wait_send()


def kernel(x, Wq, Wk, Wv, Wo):
    x2 = x.reshape(SQ, D)

    inv = 1.0 / (10000.0 ** (jnp.arange(0, DH, 2, dtype=jnp.float32) / DH))
    pos = jnp.arange(SQ, dtype=jnp.float32)[:, None] * inv[None, :]
    cos = jnp.repeat(jnp.cos(pos), 2, axis=-1)
    sin = jnp.repeat(jnp.sin(pos), 2, axis=-1)

    out = pl.pallas_call(
        _fused_body,
        out_shape=jax.ShapeDtypeStruct((SQ, D), jnp.float32),
        in_specs=[pl.BlockSpec(memory_space=pltpu.VMEM)] * 7,
        out_specs=pl.BlockSpec(memory_space=pltpu.VMEM),
        scratch_shapes=[
            pltpu.VMEM((SQ, D), jnp.bfloat16),
            pltpu.VMEM((SQ, D), jnp.bfloat16),
            pltpu.VMEM((N_DEV - 1, HALF, D), jnp.bfloat16),
            pltpu.VMEM((N_DEV - 1, HALF, D), jnp.bfloat16),
            pltpu.VMEM((N_DEV - 1, HALF, D), jnp.bfloat16),
            pltpu.VMEM((N_DEV - 1, HALF, D), jnp.bfloat16),
            pltpu.VMEM((N_DEV, HALF, D), jnp.bfloat16),
            pltpu.VMEM((N_DEV, HALF, D), jnp.bfloat16),
            pltpu.SemaphoreType.DMA((2 * (N_DEV - 1),)),
            pltpu.SemaphoreType.DMA((2 * (N_DEV - 1),)),
            pltpu.SemaphoreType.DMA((2 * (N_DEV - 1),)),
            pltpu.SemaphoreType.DMA((2 * (N_DEV - 1),)),
        ],
        compiler_params=pltpu.CompilerParams(
            collective_id=0,
            vmem_limit_bytes=64 * 1024 * 1024,
        ),
    )(x2, Wq, Wk, Wv, Wo, cos, sin)

    return out.reshape(B, SQ, D)


# device time: 117039 ns/iter; 2.6006x vs baseline; 1.0863x over previous
import math

import jax
import jax.numpy as jnp
from jax import lax
from jax.experimental import pallas as pl
from jax.experimental.pallas import tpu as pltpu

N_DEV = 4
B, SQ, D = 1, 2048, 1024
HQ_LOCAL = 8
DH = 128
NPAIR = HQ_LOCAL // 2
PW = 2 * DH
SCALE = 0.08838834764831843
CHUNK = SQ // N_DEV
HALF = CHUNK // 2


def _fused_body(
    x_ref, wq_ref, wk_ref, wv_ref, wo_ref, out_ref,
    cos_s, sin_s, kb, vb, rs_recv_cw, rs_recv_ccw, sb_cw, sb_ccw,
    ag_cw, ag_ccw, ss_cw, rs_cw, ss_ccw, rs_ccw,
):
    f32 = jnp.float32
    bf16 = jnp.bfloat16
    my = lax.axis_index("i")
    left = lax.rem(my + N_DEV - 1, N_DEV)
    right = lax.rem(my + 1, N_DEV)

    barrier_sem = pltpu.get_barrier_semaphore()
    for nbr in (left, right):
        pl.semaphore_signal(
            barrier_sem, inc=1,
            device_id=(nbr,), device_id_type=pl.DeviceIdType.MESH,
        )
    pl.semaphore_wait(barrier_sem, 2)

    row = lax.broadcasted_iota(jnp.int32, (PW, PW), 0)
    col = lax.broadcasted_iota(jnp.int32, (PW, PW), 1)
    rmat = jnp.where((col % 2 == 0) & (row == col + 1), -1.0, 0.0) + jnp.where(
        (col % 2 == 1) & (row == col - 1), 1.0, 0.0
    )

    def rope2(t, c, s):
        cc = jnp.concatenate([c, c], axis=1)
        ss = jnp.concatenate([s, s], axis=1)
        return t * cc + jnp.dot(t, rmat, preferred_element_type=f32) * ss

    rowi = lax.broadcasted_iota(jnp.int32, (SQ, DH), 0)
    coli = lax.broadcasted_iota(jnp.int32, (SQ, DH), 1)
    ang = rowi.astype(f32) * jnp.exp(
        (coli // 2).astype(f32) * (-2.0 * math.log(10000.0) / DH)
    )
    cos_s[...] = jnp.cos(ang)
    sin_s[...] = jnp.sin(ang)

    cos_all = cos_s[...]
    sin_all = sin_s[...]
    for p in range(NPAIR):
        csl = slice(p * PW, (p + 1) * PW)
        k = jnp.dot(x_ref[...], wk_ref[:, csl], preferred_element_type=f32)
        kb[:, csl] = rope2(k, cos_all, sin_all).astype(bf16)
        v = jnp.dot(x_ref[...], wv_ref[:, csl], preferred_element_type=f32)
        vb[:, csl] = v.astype(bf16)

    def compute_chunk(c, hook=None):
        rows = pl.ds(c * CHUNK, CHUNK)
        xr = x_ref[rows, :]
        cosr = cos_s[rows, :]
        sinr = sin_s[rows, :]
        acc = None
        for p in range(NPAIR):
            csl = slice(p * PW, (p + 1) * PW)
            q = jnp.dot(xr, wq_ref[:, csl], preferred_element_type=f32)
            q = rope2(q * SCALE, cosr, sinr)
            qb = q.astype(bf16)
            ctxs = []
            for j in range(2):
                hsl = slice(p * PW + j * DH, p * PW + (j + 1) * DH)
                jsl = slice(j * DH, (j + 1) * DH)
                s = lax.dot_general(
                    qb[:, jsl], kb[:, hsl], (((1,), (1,)), ((), ())),
                    preferred_element_type=f32,
                )
                w = jnp.exp(s)
                r = 1.0 / jnp.sum(w, axis=1, keepdims=True)
                ctxs.append(
                    jnp.dot(w.astype(bf16), vb[:, hsl],
                            preferred_element_type=f32) * r
                )
            part = jnp.dot(
                jnp.concatenate(ctxs, axis=1), wo_ref[csl, :],
                preferred_element_type=f32,
            )
            acc = part if acc is None else acc + part
            if hook is not None and p == 1:
                hook()
        out_ref[rows, :] = acc
        return acc

    def rdma(src, dst, ss, rs, idx, target):
        r = pltpu.make_async_remote_copy(
            src_ref=src, dst_ref=dst,
            send_sem=ss.at[idx], recv_sem=rs.at[idx],
            device_id=(target,), device_id_type=pl.DeviceIdType.MESH,
        )
        r.start()
        return r

    c0 = lax.rem(my + N_DEV - 1, N_DEV)
    c1 = lax.rem(my + 1, N_DEV)
    c2 = lax.rem(my + 2, N_DEV)

    a0 = compute_chunk(c0)
    sb_cw[0] = a0[:HALF, :].astype(bf16)
    r_cw0 = rdma(sb_cw.at[0], rs_recv_cw.at[0], ss_cw, rs_cw, 0, right)

    a1 = compute_chunk(c1)
    sb_ccw[0] = a1[HALF:, :].astype(bf16)
    r_ccw0 = rdma(sb_ccw.at[0], rs_recv_ccw.at[0], ss_ccw, rs_ccw, 0, left)

    a2 = compute_chunk(c2)
    r_cw0.wait_recv()
    sb_cw[1] = (rs_recv_cw[0].astype(f32) + a2[:HALF, :]).astype(bf16)
    r_cw1 = rdma(sb_cw.at[1], rs_recv_cw.at[1], ss_cw, rs_cw, 1, right)
    r_ccw0.wait_recv()
    sb_ccw[1] = (rs_recv_ccw[0].astype(f32) + a2[HALF:, :]).astype(bf16)
    r_ccw1 = rdma(sb_ccw.at[1], rs_recv_ccw.at[1], ss_ccw, rs_ccw, 1, left)

    h2 = {}

    def hop2_hook():
        r_cw1.wait_recv()
        sb_cw[2] = (
            rs_recv_cw[1].astype(f32) + out_ref[pl.ds(c1 * CHUNK, HALF), :]
        ).astype(bf16)
        h2["cw"] = rdma(sb_cw.at[2], rs_recv_cw.at[2], ss_cw, rs_cw, 2, right)
        r_ccw1.wait_recv()
        sb_ccw[2] = (
            rs_recv_ccw[1].astype(f32)
            + out_ref[pl.ds(c0 * CHUNK + HALF, HALF), :]
        ).astype(bf16)
        h2["ccw"] = rdma(
            sb_ccw.at[2], rs_recv_ccw.at[2], ss_ccw, rs_ccw, 2, left
        )

    a3 = compute_chunk(my, hook=hop2_hook)
    r_cw2 = h2["cw"]
    r_ccw2 = h2["ccw"]

    r_cw2.wait_recv()
    fin_low = rs_recv_cw[2].astype(f32) + a3[:HALF, :]
    out_ref[pl.ds(my * CHUNK, HALF), :] = fin_low
    ag_cw[0] = fin_low.astype(bf16)
    r_ccw2.wait_recv()
    fin_high = rs_recv_ccw[2].astype(f32) + a3[HALF:, :]
    out_ref[pl.ds(my * CHUNK + HALF, HALF), :] = fin_high
    ag_ccw[0] = fin_high.astype(bf16)

    started = [r_cw0, r_ccw0, r_cw1, r_ccw1, r_cw2, r_ccw2]
    for s in range(N_DEV - 1):
        recv_c = lax.rem(my + 2 * N_DEV - 1 - s, N_DEV)
        recv_cc = lax.rem(my + 1 + s, N_DEV)
        g1 = rdma(ag_cw.at[s], ag_cw.at[s + 1], ss_cw, rs_cw,
                  N_DEV - 1 + s, right)
        g2 = rdma(ag_ccw.at[s], ag_ccw.at[s + 1], ss_ccw, rs_ccw,
                  N_DEV - 1 + s, left)
        g1.wait_recv()
        out_ref[pl.ds(recv_c * CHUNK, HALF), :] = ag_cw[s + 1].astype(f32)
        g2.wait_recv()
        out_ref[pl.ds(recv_cc * CHUNK + HALF, HALF), :] = ag_ccw[
            s + 1
        ].astype(f32)
        started.extend([g1, g2])

    for r in started:
        r.wait_send()


def kernel(x, Wq, Wk, Wv, Wo):
    x2 = x.reshape(SQ, D)

    out = pl.pallas_call(
        _fused_body,
        out_shape=jax.ShapeDtypeStruct((SQ, D), jnp.float32),
        in_specs=[pl.BlockSpec(memory_space=pltpu.VMEM)] * 5,
        out_specs=pl.BlockSpec(memory_space=pltpu.VMEM),
        scratch_shapes=[
            pltpu.VMEM((SQ, DH), jnp.float32),
            pltpu.VMEM((SQ, DH), jnp.float32),
            pltpu.VMEM((SQ, D), jnp.bfloat16),
            pltpu.VMEM((SQ, D), jnp.bfloat16),
            pltpu.VMEM((N_DEV - 1, HALF, D), jnp.bfloat16),
            pltpu.VMEM((N_DEV - 1, HALF, D), jnp.bfloat16),
            pltpu.VMEM((N_DEV - 1, HALF, D), jnp.bfloat16),
            pltpu.VMEM((N_DEV - 1, HALF, D), jnp.bfloat16),
            pltpu.VMEM((N_DEV, HALF, D), jnp.bfloat16),
            pltpu.VMEM((N_DEV, HALF, D), jnp.bfloat16),
            pltpu.SemaphoreType.DMA((2 * (N_DEV - 1),)),
            pltpu.SemaphoreType.DMA((2 * (N_DEV - 1),)),
            pltpu.SemaphoreType.DMA((2 * (N_DEV - 1),)),
            pltpu.SemaphoreType.DMA((2 * (N_DEV - 1),)),
        ],
        compiler_params=pltpu.CompilerParams(
            collective_id=0,
            vmem_limit_bytes=64 * 1024 * 1024,
        ),
    )(x2, Wq, Wk, Wv, Wo)

    return out.reshape(B, SQ, D)


# device time: 113391 ns/iter; 2.6843x vs baseline; 1.0322x over previous
import math

import jax
import jax.numpy as jnp
from jax import lax
from jax.experimental import pallas as pl
from jax.experimental.pallas import tpu as pltpu

N_DEV = 4
B, SQ, D = 1, 2048, 1024
HQ_LOCAL = 8
DH = 128
NPAIR = HQ_LOCAL // 2
PW = 2 * DH
SCALE = 0.08838834764831843
CHUNK = SQ // N_DEV
HALF = CHUNK // 2
SUB = HALF // 2
NSEM = 3 + 2 * (N_DEV - 1)


def _fused_body(
    x_ref, wq_ref, wk_ref, wv_ref, wo_ref, out_ref,
    cos_s, sin_s, kb, vb, rs_recv_cw, rs_recv_ccw, sb_cw, sb_ccw,
    ag_cw, ag_ccw, ss_cw, rs_cw, ss_ccw, rs_ccw,
):
    f32 = jnp.float32
    bf16 = jnp.bfloat16
    my = lax.axis_index("i")
    left = lax.rem(my + N_DEV - 1, N_DEV)
    right = lax.rem(my + 1, N_DEV)

    barrier_sem = pltpu.get_barrier_semaphore()
    for nbr in (left, right):
        pl.semaphore_signal(
            barrier_sem, inc=1,
            device_id=(nbr,), device_id_type=pl.DeviceIdType.MESH,
        )
    pl.semaphore_wait(barrier_sem, 2)

    row = lax.broadcasted_iota(jnp.int32, (PW, PW), 0)
    col = lax.broadcasted_iota(jnp.int32, (PW, PW), 1)
    rmat = jnp.where((col % 2 == 0) & (row == col + 1), -1.0, 0.0) + jnp.where(
        (col % 2 == 1) & (row == col - 1), 1.0, 0.0
    )

    def rope2(t, c, s):
        cc = jnp.concatenate([c, c], axis=1)
        ss = jnp.concatenate([s, s], axis=1)
        return t * cc + jnp.dot(t, rmat, preferred_element_type=f32) * ss

    rowi = lax.broadcasted_iota(jnp.int32, (SQ, DH), 0)
    coli = lax.broadcasted_iota(jnp.int32, (SQ, DH), 1)
    ang = rowi.astype(f32) * jnp.exp(
        (coli // 2).astype(f32) * (-2.0 * math.log(10000.0) / DH)
    )
    cos_s[...] = jnp.cos(ang)
    sin_s[...] = jnp.sin(ang)

    cos_all = cos_s[...]
    sin_all = sin_s[...]
    for p in range(NPAIR):
        csl = slice(p * PW, (p + 1) * PW)
        k = jnp.dot(x_ref[...], wk_ref[:, csl], preferred_element_type=f32)
        kb[:, csl] = rope2(k, cos_all, sin_all).astype(bf16)
        v = jnp.dot(x_ref[...], wv_ref[:, csl], preferred_element_type=f32)
        vb[:, csl] = v.astype(bf16)

    def compute_chunk(c, hook=None):
        rows = pl.ds(c * CHUNK, CHUNK)
        xr = x_ref[rows, :]
        cosr = cos_s[rows, :]
        sinr = sin_s[rows, :]
        acc = None
        for p in range(NPAIR):
            csl = slice(p * PW, (p + 1) * PW)
            q = jnp.dot(xr, wq_ref[:, csl], preferred_element_type=f32)
            q = rope2(q * SCALE, cosr, sinr)
            qb = q.astype(bf16)
            ctxs = []
            for j in range(2):
                hsl = slice(p * PW + j * DH, p * PW + (j + 1) * DH)
                jsl = slice(j * DH, (j + 1) * DH)
                s = lax.dot_general(
                    qb[:, jsl], kb[:, hsl], (((1,), (1,)), ((), ())),
                    preferred_element_type=f32,
                )
                w = jnp.exp(s)
                r = 1.0 / jnp.sum(w, axis=1, keepdims=True)
                ctxs.append(
                    jnp.dot(w.astype(bf16), vb[:, hsl],
                            preferred_element_type=f32) * r
                )
            part = jnp.dot(
                jnp.concatenate(ctxs, axis=1), wo_ref[csl, :],
                preferred_element_type=f32,
            )
            acc = part if acc is None else acc + part
            if hook is not None and p == 1:
                hook()
        out_ref[rows, :] = acc
        return acc

    def rdma(src, dst, ss, rs, idx, target):
        r = pltpu.make_async_remote_copy(
            src_ref=src, dst_ref=dst,
            send_sem=ss.at[idx], recv_sem=rs.at[idx],
            device_id=(target,), device_id_type=pl.DeviceIdType.MESH,
        )
        r.start()
        return r

    c0 = lax.rem(my + N_DEV - 1, N_DEV)
    c1 = lax.rem(my + 1, N_DEV)
    c2 = lax.rem(my + 2, N_DEV)

    a0 = compute_chunk(c0)
    sb_cw[0] = a0[:HALF, :].astype(bf16)
    r_cw0 = rdma(sb_cw.at[0], rs_recv_cw.at[0], ss_cw, rs_cw, 0, right)

    a1 = compute_chunk(c1)
    sb_ccw[0] = a1[HALF:, :].astype(bf16)
    r_ccw0 = rdma(sb_ccw.at[0], rs_recv_ccw.at[0], ss_ccw, rs_ccw, 0, left)

    a2 = compute_chunk(c2)
    r_cw0.wait_recv()
    sb_cw[1] = (rs_recv_cw[0].astype(f32) + a2[:HALF, :]).astype(bf16)
    r_cw1 = rdma(sb_cw.at[1], rs_recv_cw.at[1], ss_cw, rs_cw, 1, right)
    r_ccw0.wait_recv()
    sb_ccw[1] = (rs_recv_ccw[0].astype(f32) + a2[HALF:, :]).astype(bf16)
    r_ccw1 = rdma(sb_ccw.at[1], rs_recv_ccw.at[1], ss_ccw, rs_ccw, 1, left)

    h2 = {}

    def hop2_hook():
        r_cw1.wait_recv()
        sb_cw[2] = (
            rs_recv_cw[1].astype(f32) + out_ref[pl.ds(c1 * CHUNK, HALF), :]
        ).astype(bf16)
        h2["cw"] = rdma(sb_cw.at[2], rs_recv_cw.at[2], ss_cw, rs_cw, 2, right)
        r_ccw1.wait_recv()
        sb_ccw[2] = (
            rs_recv_ccw[1].astype(f32)
            + out_ref[pl.ds(c0 * CHUNK + HALF, HALF), :]
        ).astype(bf16)
        h2["ccw"] = rdma(
            sb_ccw.at[2], rs_recv_ccw.at[2], ss_ccw, rs_ccw, 2, left
        )

    a3 = compute_chunk(my, hook=hop2_hook)
    r_cw2 = h2["cw"]
    r_ccw2 = h2["ccw"]

    r_cw2.wait_recv()
    fin_low = rs_recv_cw[2].astype(f32) + a3[:HALF, :]
    out_ref[pl.ds(my * CHUNK, HALF), :] = fin_low
    ag_cw[0] = fin_low[:SUB, :].astype(bf16)
    ag_cw[1] = fin_low[SUB:, :].astype(bf16)
    r_ccw2.wait_recv()
    fin_high = rs_recv_ccw[2].astype(f32) + a3[HALF:, :]
    out_ref[pl.ds(my * CHUNK + HALF, HALF), :] = fin_high
    ag_ccw[0] = fin_high[:SUB, :].astype(bf16)
    ag_ccw[1] = fin_high[SUB:, :].astype(bf16)

    started = [r_cw0, r_ccw0, r_cw1, r_ccw1, r_cw2, r_ccw2]
    g_cw = {}
    g_ccw = {}
    for u in range(2):
        g_cw[(0, u)] = rdma(
            ag_cw.at[u], ag_cw.at[2 + u], ss_cw, rs_cw, 3 + u, right
        )
        g_ccw[(0, u)] = rdma(
            ag_ccw.at[u], ag_ccw.at[2 + u], ss_ccw, rs_ccw, 3 + u, left
        )
    for s in range(1, N_DEV - 1):
        rc = lax.rem(my + 2 * N_DEV - s, N_DEV)
        rcc = lax.rem(my + s, N_DEV)
        for u in range(2):
            g_cw[(s - 1, u)].wait_recv()
            g_cw[(s, u)] = rdma(
                ag_cw.at[2 * s + u], ag_cw.at[2 * (s + 1) + u],
                ss_cw, rs_cw, 3 + 2 * s + u, right,
            )
            out_ref[pl.ds(rc * CHUNK + u * SUB, SUB), :] = ag_cw[
                2 * s + u
            ].astype(f32)
            g_ccw[(s - 1, u)].wait_recv()
            g_ccw[(s, u)] = rdma(
                ag_ccw.at[2 * s + u], ag_ccw.at[2 * (s + 1) + u],
                ss_ccw, rs_ccw, 3 + 2 * s + u, left,
            )
            out_ref[pl.ds(rcc * CHUNK + HALF + u * SUB, SUB), :] = ag_ccw[
                2 * s + u
            ].astype(f32)
    rc = lax.rem(my + N_DEV + 1, N_DEV)
    rcc = lax.rem(my + N_DEV - 1, N_DEV)
    for u in range(2):
        g_cw[(N_DEV - 2, u)].wait_recv()
        out_ref[pl.ds(rc * CHUNK + u * SUB, SUB), :] = ag_cw[
            2 * (N_DEV - 1) + u
        ].astype(f32)
        g_ccw[(N_DEV - 2, u)].wait_recv()
        out_ref[pl.ds(rcc * CHUNK + HALF + u * SUB, SUB), :] = ag_ccw[
            2 * (N_DEV - 1) + u
        ].astype(f32)
    started.extend(g_cw.values())
    started.extend(g_ccw.values())

    for r in started:
        r.wait_send()


def kernel(x, Wq, Wk, Wv, Wo):
    x2 = x.reshape(SQ, D)

    out = pl.pallas_call(
        _fused_body,
        out_shape=jax.ShapeDtypeStruct((SQ, D), jnp.float32),
        in_specs=[pl.BlockSpec(memory_space=pltpu.VMEM)] * 5,
        out_specs=pl.BlockSpec(memory_space=pltpu.VMEM),
        scratch_shapes=[
            pltpu.VMEM((SQ, DH), jnp.float32),
            pltpu.VMEM((SQ, DH), jnp.float32),
            pltpu.VMEM((SQ, D), jnp.bfloat16),
            pltpu.VMEM((SQ, D), jnp.bfloat16),
            pltpu.VMEM((N_DEV - 1, HALF, D), jnp.bfloat16),
            pltpu.VMEM((N_DEV - 1, HALF, D), jnp.bfloat16),
            pltpu.VMEM((N_DEV - 1, HALF, D), jnp.bfloat16),
            pltpu.VMEM((N_DEV - 1, HALF, D), jnp.bfloat16),
            pltpu.VMEM((2 * N_DEV, SUB, D), jnp.bfloat16),
            pltpu.VMEM((2 * N_DEV, SUB, D), jnp.bfloat16),
            pltpu.SemaphoreType.DMA((NSEM,)),
            pltpu.SemaphoreType.DMA((NSEM,)),
            pltpu.SemaphoreType.DMA((NSEM,)),
            pltpu.SemaphoreType.DMA((NSEM,)),
        ],
        compiler_params=pltpu.CompilerParams(
            collective_id=0,
            vmem_limit_bytes=64 * 1024 * 1024,
        ),
    )(x2, Wq, Wk, Wv, Wo)

    return out.reshape(B, SQ, D)
